# Initial kernel scaffold; baseline (speedup 1.0000x reference)
import jax
import jax.numpy as jnp
from jax.experimental import pallas as pl


def kernel(mol_x, pro_x, mol_edge_index, pro_edge_index, mol_batch, pro_batch, mol_W0, mol_b0, mol_W1s, mol_W2s, mol_cbs, mol_Wout, mol_bout, pro_W0, pro_b0, pro_W1s, pro_W2s, pro_cbs, pro_Wout, pro_bout, fc1_W, fc1_b, fc2_W, fc2_b, out_W, out_b):
    raise NotImplementedError("write your pallas kernel here")


# speedup vs baseline: 7.8985x; 7.8985x over previous
import functools

import jax
import jax.numpy as jnp
from jax import lax
from jax.experimental import pallas as pl
from jax.experimental.pallas import tpu as pltpu
from jax.experimental.pallas import tpu_sc as plsc

L = 3
ALPHA = 0.2
NSUB = 16
LANE = 128
GROUP = 16
TILE = 128
DEGW = 16


def _sc_mesh():
    return plsc.VectorSubcoreMesh(core_axis_name="c", subcore_axis_name="s")


def _deg_kernel(npad, ngroups):
    tiles = npad // (NSUB * TILE)

    @functools.partial(
        pl.kernel,
        out_type=jax.ShapeDtypeStruct((2, npad, 128), jnp.float32),
        mesh=_sc_mesh(),
        scratch_types=[
            pltpu.VMEM_SHARED((npad + 8, 128), jnp.float32),
            pltpu.VMEM((GROUP, LANE), jnp.int32),
            pltpu.VMEM((LANE, 128), jnp.float32),
        ],
    )
    def deg_kernel(ones_hbm, dstpad, deg_out, deg_sp, dst_v, rows_v):
        c = lax.axis_index("c")
        s = lax.axis_index("s")
        base = s * (tiles * TILE)
        pltpu.sync_copy(ones_hbm, rows_v)
        for k in range(tiles):
            pltpu.sync_copy(rows_v, deg_sp.at[pl.ds(base + k * TILE, TILE)])
        plsc.subcore_barrier()

        def body(grp, carry):
            pltpu.sync_copy(dstpad.at[c, s, grp], dst_v)
            for j in range(GROUP):
                pltpu.sync_copy(rows_v, deg_sp.at[dst_v.at[j]], add=True)
            return carry

        lax.fori_loop(0, ngroups, body, 0)
        plsc.subcore_barrier()
        for k in range(tiles):
            pltpu.sync_copy(deg_sp.at[pl.ds(base + k * TILE, TILE)], rows_v)
            pltpu.sync_copy(rows_v, deg_out.at[c, pl.ds(base + k * TILE, TILE)])

    return deg_kernel


def _prop_kernel(npad, ngroups):
    tiles = npad // (NSUB * TILE)

    @functools.partial(
        pl.kernel,
        out_type=jax.ShapeDtypeStruct((2, npad, 128), jnp.float32),
        mesh=_sc_mesh(),
        scratch_types=[
            pltpu.VMEM_SHARED((npad + 8, 128), jnp.float32),
            pltpu.VMEM((GROUP, LANE), jnp.int32),
            pltpu.VMEM((GROUP, LANE), jnp.int32),
            pltpu.VMEM((LANE, 128), jnp.float32),
            pltpu.SemaphoreType.DMA,
        ],
    )
    def prop_kernel(gflat, srcpad, dstpad, t_out, acc, src_v, dst_v, rows_v, sem):
        c = lax.axis_index("c")
        s = lax.axis_index("s")
        base = s * (tiles * TILE)
        for k in range(tiles):
            pltpu.sync_copy(gflat.at[pl.ds(c * npad + base + k * TILE, TILE)], rows_v)
            pltpu.sync_copy(rows_v, acc.at[pl.ds(base + k * TILE, TILE)])
        plsc.subcore_barrier()

        def body(grp, carry):
            pltpu.sync_copy(srcpad.at[c, s, grp], src_v)
            pltpu.sync_copy(dstpad.at[c, s, grp], dst_v)
            for j in range(GROUP):
                pltpu.async_copy(gflat.at[src_v.at[j]], rows_v, sem).wait()
                pltpu.sync_copy(rows_v, acc.at[dst_v.at[j]], add=True)
            return carry

        lax.fori_loop(0, ngroups, body, 0)
        plsc.subcore_barrier()
        for k in range(tiles):
            pltpu.sync_copy(acc.at[pl.ds(base + k * TILE, TILE)], rows_v)
            pltpu.sync_copy(rows_v, t_out.at[c, pl.ds(base + k * TILE, TILE)])

    return prop_kernel


_F32 = jnp.float32


def _init_tc(x, w0, b0, deg, blk):
    n = x.shape[1]
    grid = (2, n // blk)
    out = [
        jax.ShapeDtypeStruct((2, n, 128), _F32),
        jax.ShapeDtypeStruct((2, n, 128), _F32),
        jax.ShapeDtypeStruct((2, n, 1), _F32),
    ]

    def body(x_ref, w_ref, b_ref, deg_ref, h_ref, g_ref, dinv_ref):
        h = jnp.dot(x_ref[0], w_ref[0], preferred_element_type=_F32) + b_ref[0]
        h = jnp.maximum(h, 0.0)
        dinv = lax.rsqrt(deg_ref[0][:, 0:1])
        h_ref[0] = h
        g_ref[0] = h * dinv
        dinv_ref[0] = dinv

    return pl.pallas_call(
        body,
        grid=grid,
        in_specs=[
            pl.BlockSpec((1, blk, 128), lambda c, b: (c, b, 0)),
            pl.BlockSpec((1, 128, 128), lambda c, b: (c, 0, 0)),
            pl.BlockSpec((1, 1, 128), lambda c, b: (c, 0, 0)),
            pl.BlockSpec((1, blk, 128), lambda c, b: (c, b, 0)),
        ],
        out_specs=[
            pl.BlockSpec((1, blk, 128), lambda c, b: (c, b, 0)),
            pl.BlockSpec((1, blk, 128), lambda c, b: (c, b, 0)),
            pl.BlockSpec((1, blk, 1), lambda c, b: (c, b, 0)),
        ],
        out_shape=out,
    )(x, w0, b0, deg)


def _layer_tc(t, h0, h, dinv, w1, w2, cb, blk):
    n = t.shape[1]
    grid = (2, n // blk)
    out = [
        jax.ShapeDtypeStruct((2, n, 128), _F32),
        jax.ShapeDtypeStruct((2, n, 128), _F32),
    ]

    def body(t_ref, h0_ref, h_ref, dinv_ref, w1_ref, w2_ref, cb_ref, hn_ref, gn_ref):
        dinv = dinv_ref[0]
        agg = t_ref[0] * dinv
        o = (
            jnp.dot(agg, w1_ref[0], preferred_element_type=_F32)
            + jnp.dot(h0_ref[0], w2_ref[0], preferred_element_type=_F32)
            + cb_ref[0]
        )
        hn = jnp.maximum(o, 0.0) + h_ref[0]
        hn_ref[0] = hn
        gn_ref[0] = hn * dinv

    return pl.pallas_call(
        body,
        grid=grid,
        in_specs=[
            pl.BlockSpec((1, blk, 128), lambda c, b: (c, b, 0)),
            pl.BlockSpec((1, blk, 128), lambda c, b: (c, b, 0)),
            pl.BlockSpec((1, blk, 128), lambda c, b: (c, b, 0)),
            pl.BlockSpec((1, blk, 1), lambda c, b: (c, b, 0)),
            pl.BlockSpec((1, 128, 128), lambda c, b: (c, 0, 0)),
            pl.BlockSpec((1, 128, 128), lambda c, b: (c, 0, 0)),
            pl.BlockSpec((1, 1, 128), lambda c, b: (c, 0, 0)),
        ],
        out_specs=[
            pl.BlockSpec((1, blk, 128), lambda c, b: (c, b, 0)),
            pl.BlockSpec((1, blk, 128), lambda c, b: (c, b, 0)),
        ],
        out_shape=out,
    )(t, h0, h, dinv, w1, w2, cb)


def _head_tc(h, wout, bout, fc1w, fc1b, fc2w, fc2b, outw, outb, n, blk):
    grid = (n // blk,)

    def body(h_ref, wout_ref, bout_ref, f1w_ref, f1b_ref, f2w_ref, f2b_ref,
             ow_ref, ob_ref, o_ref):
        ym = jnp.dot(h_ref[0], wout_ref[0], preferred_element_type=_F32) + bout_ref[0]
        yp = jnp.dot(h_ref[1], wout_ref[1], preferred_element_type=_F32) + bout_ref[1]
        t1 = (
            jnp.dot(ym, f1w_ref[0:128, :], preferred_element_type=_F32)
            + jnp.dot(yp, f1w_ref[128:256, :], preferred_element_type=_F32)
            + f1b_ref[...]
        )
        t1 = jnp.maximum(t1, 0.0)
        t2 = jnp.maximum(
            jnp.dot(t1, f2w_ref[...], preferred_element_type=_F32) + f2b_ref[...], 0.0)
        o_ref[...] = jnp.dot(t2, ow_ref[...], preferred_element_type=_F32) + ob_ref[...]

    return pl.pallas_call(
        body,
        grid=grid,
        in_specs=[
            pl.BlockSpec((2, blk, 128), lambda b: (0, b, 0)),
            pl.BlockSpec((2, 128, 128), lambda b: (0, 0, 0)),
            pl.BlockSpec((2, 1, 128), lambda b: (0, 0, 0)),
            pl.BlockSpec((256, 1024), lambda b: (0, 0)),
            pl.BlockSpec((1, 1024), lambda b: (0, 0)),
            pl.BlockSpec((1024, 512), lambda b: (0, 0)),
            pl.BlockSpec((1, 512), lambda b: (0, 0)),
            pl.BlockSpec((512, 1), lambda b: (0, 0)),
            pl.BlockSpec((1, 1), lambda b: (0, 0)),
        ],
        out_specs=pl.BlockSpec((blk, 1), lambda b: (b, 0)),
        out_shape=jax.ShapeDtypeStruct((n, 1), _F32),
    )(h, wout, bout, fc1w, fc1b, fc2w, fc2b, outw, outb)


def _prep_edges(ei, npad, offset, ngroups):
    src = ei[0].astype(jnp.int32)
    dst = ei[1].astype(jnp.int32)
    e = src.shape[0]
    total = NSUB * ngroups * GROUP * LANE
    pad = total - e
    src = jnp.concatenate([src + offset, jnp.full((pad,), offset, jnp.int32)])
    dst = jnp.concatenate([dst, jnp.full((pad,), npad, jnp.int32)])
    return (src.reshape(NSUB, ngroups, GROUP, LANE),
            dst.reshape(NSUB, ngroups, GROUP, LANE))


def _pad_feat(x, w, npad):
    n, f = x.shape
    x = jnp.pad(x, ((0, npad - n), (0, 128 - f)))
    w = jnp.pad(w, ((0, 128 - f), (0, 0)))
    return x, w


def kernel(mol_x, pro_x, mol_edge_index, pro_edge_index, mol_batch, pro_batch,
           mol_W0, mol_b0, mol_W1s, mol_W2s, mol_cbs, mol_Wout, mol_bout,
           pro_W0, pro_b0, pro_W1s, pro_W2s, pro_cbs, pro_Wout, pro_bout,
           fc1_W, fc1_b, fc2_W, fc2_b, out_W, out_b):
    n = mol_x.shape[0]
    e = mol_edge_index.shape[1]
    npad = -(-n // (NSUB * TILE)) * NSUB * TILE
    nchunks = -(-e // (NSUB * LANE))
    ngroups = -(-nchunks // GROUP)
    nchunks_pad = ngroups * GROUP
    blk = npad // 8
    blk_head = 1000 if n % 1000 == 0 else n

    mx, mW0 = _pad_feat(mol_x, mol_W0, npad)
    px, pW0 = _pad_feat(pro_x, pro_W0, npad)
    x = jnp.stack([mx, px])
    w0 = jnp.stack([mW0, pW0])
    b0 = jnp.stack([mol_b0, pro_b0])[:, None, :]

    msrc, mdst = _prep_edges(mol_edge_index, npad, 0, ngroups)
    psrc, pdst = _prep_edges(pro_edge_index, npad, npad, ngroups)
    srcpad = jnp.stack([msrc, psrc])
    dstpad = jnp.stack([mdst, pdst])

    w1 = jnp.stack([mol_W1s, pro_W1s]) * (1.0 - ALPHA)
    w2 = jnp.stack([mol_W2s, pro_W2s]) * ALPHA
    cb = jnp.stack([mol_cbs, pro_cbs])[:, :, None, :]
    wout = jnp.stack([mol_Wout, pro_Wout])
    bout = jnp.stack([mol_bout, pro_bout])[:, None, :]
    f1b = fc1_b[None, :]
    f2b = fc2_b[None, :]
    ob = out_b[None, :]

    ones_tile = jnp.ones((LANE, 128), jnp.float32)

    deg128 = _deg_kernel(npad, ngroups)(ones_tile, dstpad)
    h0, g, dinv = _init_tc(x, w0, b0, deg128, blk)

    prop = _prop_kernel(npad, ngroups)
    h = h0
    for i in range(L):
        t = prop(g.reshape(2 * npad, 128), srcpad, dstpad)
        h, g = _layer_tc(t, h0, h, dinv, w1[:, i], w2[:, i], cb[:, i], blk)

    return _head_tc(h, wout, bout, fc1_W, f1b, fc2_W, f2b, out_W, ob, n, blk_head)


# speedup vs baseline: 8.8466x; 1.1200x over previous
import functools

import jax
import jax.numpy as jnp
from jax import lax
from jax.experimental import pallas as pl
from jax.experimental.pallas import tpu as pltpu
from jax.experimental.pallas import tpu_sc as plsc

L = 3
ALPHA = 0.2
NSUB = 16
LANE = 128
GROUP = 16
TILE = 128
DEGW = 16


def _sc_mesh():
    return plsc.VectorSubcoreMesh(core_axis_name="c", subcore_axis_name="s")


def _deg_kernel(npad, ngroups):
    tiles = npad // (NSUB * TILE)

    @functools.partial(
        pl.kernel,
        out_type=jax.ShapeDtypeStruct((2, npad, 128), jnp.float32),
        mesh=_sc_mesh(),
        scratch_types=[
            pltpu.VMEM_SHARED((npad + 8, 128), jnp.float32),
            pltpu.VMEM((GROUP, LANE), jnp.int32),
            pltpu.VMEM((LANE, 128), jnp.float32),
            pltpu.SemaphoreType.DMA,
        ],
    )
    def deg_kernel(ones_hbm, dstpad, deg_out, deg_sp, dst_v, rows_v, sem):
        c = lax.axis_index("c")
        s = lax.axis_index("s")
        base = s * (tiles * TILE)
        pltpu.sync_copy(ones_hbm, rows_v)
        for k in range(tiles):
            pltpu.sync_copy(rows_v, deg_sp.at[pl.ds(base + k * TILE, TILE)])
        plsc.subcore_barrier()

        def body(grp, carry):
            pltpu.sync_copy(dstpad.at[c, s, grp], dst_v)
            ds_ = [pltpu.async_copy(rows_v, deg_sp.at[dst_v.at[j]], sem,
                                    add=True) for j in range(GROUP)]
            for d in ds_:
                d.wait()
            return carry

        lax.fori_loop(0, ngroups, body, 0)
        plsc.subcore_barrier()
        for k in range(tiles):
            pltpu.sync_copy(deg_sp.at[pl.ds(base + k * TILE, TILE)], rows_v)
            pltpu.sync_copy(rows_v, deg_out.at[c, pl.ds(base + k * TILE, TILE)])

    return deg_kernel


def _prop_kernel(npad, ngroups):
    tiles = npad // (NSUB * TILE)

    @functools.partial(
        pl.kernel,
        out_type=jax.ShapeDtypeStruct((2, npad, 128), jnp.float32),
        mesh=_sc_mesh(),
        scratch_types=[
            pltpu.VMEM_SHARED((npad + 8, 128), jnp.float32),
            pltpu.VMEM((GROUP, LANE), jnp.int32),
            pltpu.VMEM((GROUP, LANE), jnp.int32),
            pltpu.VMEM((LANE, 128), jnp.float32),
            pltpu.VMEM((LANE, 128), jnp.float32),
            pltpu.SemaphoreType.DMA,
            pltpu.SemaphoreType.DMA,
            pltpu.SemaphoreType.DMA,
            pltpu.SemaphoreType.DMA,
        ],
    )
    def prop_kernel(gflat, srcpad, dstpad, t_out, acc, src_v, dst_v,
                    rows0, rows1, gsem0, gsem1, ssem0, ssem1):
        c = lax.axis_index("c")
        s = lax.axis_index("s")
        base = s * (tiles * TILE)
        rows = (rows0, rows1)
        gsem = (gsem0, gsem1)
        ssem = (ssem0, ssem1)

        ind = [None] * tiles
        outd = [None] * tiles
        ind[0] = pltpu.async_copy(
            gflat.at[pl.ds(c * npad + base, TILE)], rows[0], gsem[0])
        for k in range(tiles):
            ind[k].wait()
            if k >= 1:
                outd[k - 1].wait()
            if k + 1 < tiles:
                b = (k + 1) % 2
                ind[k + 1] = pltpu.async_copy(
                    gflat.at[pl.ds(c * npad + base + (k + 1) * TILE, TILE)],
                    rows[b], gsem[b])
            b = k % 2
            outd[k] = pltpu.async_copy(
                rows[b], acc.at[pl.ds(base + k * TILE, TILE)], ssem[b])
        outd[tiles - 1].wait()
        plsc.subcore_barrier()

        def body(grp, carry):
            pltpu.sync_copy(srcpad.at[c, s, grp], src_v)
            pltpu.sync_copy(dstpad.at[c, s, grp], dst_v)
            gd = [None] * GROUP
            sd = [None] * GROUP
            gd[0] = pltpu.async_copy(gflat.at[src_v.at[0]], rows[0], gsem[0])
            for j in range(GROUP):
                gd[j].wait()
                if j >= 1:
                    sd[j - 1].wait()
                if j + 1 < GROUP:
                    b = (j + 1) % 2
                    gd[j + 1] = pltpu.async_copy(
                        gflat.at[src_v.at[j + 1]], rows[b], gsem[b])
                b = j % 2
                sd[j] = pltpu.async_copy(
                    rows[b], acc.at[dst_v.at[j]], ssem[b], add=True)
            sd[GROUP - 1].wait()
            return carry

        lax.fori_loop(0, ngroups, body, 0)
        plsc.subcore_barrier()

        ind = [None] * tiles
        outd = [None] * tiles
        ind[0] = pltpu.async_copy(acc.at[pl.ds(base, TILE)], rows[0], gsem[0])
        for k in range(tiles):
            ind[k].wait()
            if k >= 1:
                outd[k - 1].wait()
            if k + 1 < tiles:
                b = (k + 1) % 2
                ind[k + 1] = pltpu.async_copy(
                    acc.at[pl.ds(base + (k + 1) * TILE, TILE)], rows[b], gsem[b])
            b = k % 2
            outd[k] = pltpu.async_copy(
                rows[b], t_out.at[c, pl.ds(base + k * TILE, TILE)], ssem[b])
        outd[tiles - 1].wait()

    return prop_kernel


_F32 = jnp.float32


def _init_tc(x, w0, b0, deg, blk):
    n = x.shape[1]
    grid = (2, n // blk)
    out = [
        jax.ShapeDtypeStruct((2, n, 128), _F32),
        jax.ShapeDtypeStruct((2, n, 128), _F32),
        jax.ShapeDtypeStruct((2, n, 1), _F32),
    ]

    def body(x_ref, w_ref, b_ref, deg_ref, h_ref, g_ref, dinv_ref):
        h = jnp.dot(x_ref[0], w_ref[0], preferred_element_type=_F32) + b_ref[0]
        h = jnp.maximum(h, 0.0)
        dinv = lax.rsqrt(deg_ref[0][:, 0:1])
        h_ref[0] = h
        g_ref[0] = h * dinv
        dinv_ref[0] = dinv

    return pl.pallas_call(
        body,
        grid=grid,
        in_specs=[
            pl.BlockSpec((1, blk, 128), lambda c, b: (c, b, 0)),
            pl.BlockSpec((1, 128, 128), lambda c, b: (c, 0, 0)),
            pl.BlockSpec((1, 1, 128), lambda c, b: (c, 0, 0)),
            pl.BlockSpec((1, blk, 128), lambda c, b: (c, b, 0)),
        ],
        out_specs=[
            pl.BlockSpec((1, blk, 128), lambda c, b: (c, b, 0)),
            pl.BlockSpec((1, blk, 128), lambda c, b: (c, b, 0)),
            pl.BlockSpec((1, blk, 1), lambda c, b: (c, b, 0)),
        ],
        out_shape=out,
    )(x, w0, b0, deg)


def _layer_tc(t, h0, h, dinv, w1, w2, cb, blk):
    n = t.shape[1]
    grid = (2, n // blk)
    out = [
        jax.ShapeDtypeStruct((2, n, 128), _F32),
        jax.ShapeDtypeStruct((2, n, 128), _F32),
    ]

    def body(t_ref, h0_ref, h_ref, dinv_ref, w1_ref, w2_ref, cb_ref, hn_ref, gn_ref):
        dinv = dinv_ref[0]
        agg = t_ref[0] * dinv
        o = (
            jnp.dot(agg, w1_ref[0], preferred_element_type=_F32)
            + jnp.dot(h0_ref[0], w2_ref[0], preferred_element_type=_F32)
            + cb_ref[0]
        )
        hn = jnp.maximum(o, 0.0) + h_ref[0]
        hn_ref[0] = hn
        gn_ref[0] = hn * dinv

    return pl.pallas_call(
        body,
        grid=grid,
        in_specs=[
            pl.BlockSpec((1, blk, 128), lambda c, b: (c, b, 0)),
            pl.BlockSpec((1, blk, 128), lambda c, b: (c, b, 0)),
            pl.BlockSpec((1, blk, 128), lambda c, b: (c, b, 0)),
            pl.BlockSpec((1, blk, 1), lambda c, b: (c, b, 0)),
            pl.BlockSpec((1, 128, 128), lambda c, b: (c, 0, 0)),
            pl.BlockSpec((1, 128, 128), lambda c, b: (c, 0, 0)),
            pl.BlockSpec((1, 1, 128), lambda c, b: (c, 0, 0)),
        ],
        out_specs=[
            pl.BlockSpec((1, blk, 128), lambda c, b: (c, b, 0)),
            pl.BlockSpec((1, blk, 128), lambda c, b: (c, b, 0)),
        ],
        out_shape=out,
    )(t, h0, h, dinv, w1, w2, cb)


def _head_tc(h, wout, bout, fc1w, fc1b, fc2w, fc2b, outw, outb, n, blk):
    grid = (n // blk,)

    def body(h_ref, wout_ref, bout_ref, f1w_ref, f1b_ref, f2w_ref, f2b_ref,
             ow_ref, ob_ref, o_ref):
        ym = jnp.dot(h_ref[0], wout_ref[0], preferred_element_type=_F32) + bout_ref[0]
        yp = jnp.dot(h_ref[1], wout_ref[1], preferred_element_type=_F32) + bout_ref[1]
        t1 = (
            jnp.dot(ym, f1w_ref[0:128, :], preferred_element_type=_F32)
            + jnp.dot(yp, f1w_ref[128:256, :], preferred_element_type=_F32)
            + f1b_ref[...]
        )
        t1 = jnp.maximum(t1, 0.0)
        t2 = jnp.maximum(
            jnp.dot(t1, f2w_ref[...], preferred_element_type=_F32) + f2b_ref[...], 0.0)
        o_ref[...] = jnp.dot(t2, ow_ref[...], preferred_element_type=_F32) + ob_ref[...]

    return pl.pallas_call(
        body,
        grid=grid,
        in_specs=[
            pl.BlockSpec((2, blk, 128), lambda b: (0, b, 0)),
            pl.BlockSpec((2, 128, 128), lambda b: (0, 0, 0)),
            pl.BlockSpec((2, 1, 128), lambda b: (0, 0, 0)),
            pl.BlockSpec((256, 1024), lambda b: (0, 0)),
            pl.BlockSpec((1, 1024), lambda b: (0, 0)),
            pl.BlockSpec((1024, 512), lambda b: (0, 0)),
            pl.BlockSpec((1, 512), lambda b: (0, 0)),
            pl.BlockSpec((512, 1), lambda b: (0, 0)),
            pl.BlockSpec((1, 1), lambda b: (0, 0)),
        ],
        out_specs=pl.BlockSpec((blk, 1), lambda b: (b, 0)),
        out_shape=jax.ShapeDtypeStruct((n, 1), _F32),
    )(h, wout, bout, fc1w, fc1b, fc2w, fc2b, outw, outb)


def _prep_edges(ei, npad, offset, ngroups):
    src = ei[0].astype(jnp.int32)
    dst = ei[1].astype(jnp.int32)
    e = src.shape[0]
    total = NSUB * ngroups * GROUP * LANE
    pad = total - e
    src = jnp.concatenate([src + offset, jnp.full((pad,), offset, jnp.int32)])
    dst = jnp.concatenate([dst, jnp.full((pad,), npad, jnp.int32)])
    return (src.reshape(NSUB, ngroups, GROUP, LANE),
            dst.reshape(NSUB, ngroups, GROUP, LANE))


def _pad_feat(x, w, npad):
    n, f = x.shape
    x = jnp.pad(x, ((0, npad - n), (0, 128 - f)))
    w = jnp.pad(w, ((0, 128 - f), (0, 0)))
    return x, w


def kernel(mol_x, pro_x, mol_edge_index, pro_edge_index, mol_batch, pro_batch,
           mol_W0, mol_b0, mol_W1s, mol_W2s, mol_cbs, mol_Wout, mol_bout,
           pro_W0, pro_b0, pro_W1s, pro_W2s, pro_cbs, pro_Wout, pro_bout,
           fc1_W, fc1_b, fc2_W, fc2_b, out_W, out_b):
    n = mol_x.shape[0]
    e = mol_edge_index.shape[1]
    npad = -(-n // (NSUB * TILE)) * NSUB * TILE
    nchunks = -(-e // (NSUB * LANE))
    ngroups = -(-nchunks // GROUP)
    nchunks_pad = ngroups * GROUP
    blk = npad // 8
    blk_head = 1000 if n % 1000 == 0 else n

    mx, mW0 = _pad_feat(mol_x, mol_W0, npad)
    px, pW0 = _pad_feat(pro_x, pro_W0, npad)
    x = jnp.stack([mx, px])
    w0 = jnp.stack([mW0, pW0])
    b0 = jnp.stack([mol_b0, pro_b0])[:, None, :]

    msrc, mdst = _prep_edges(mol_edge_index, npad, 0, ngroups)
    psrc, pdst = _prep_edges(pro_edge_index, npad, npad, ngroups)
    srcpad = jnp.stack([msrc, psrc])
    dstpad = jnp.stack([mdst, pdst])

    w1 = jnp.stack([mol_W1s, pro_W1s]) * (1.0 - ALPHA)
    w2 = jnp.stack([mol_W2s, pro_W2s]) * ALPHA
    cb = jnp.stack([mol_cbs, pro_cbs])[:, :, None, :]
    wout = jnp.stack([mol_Wout, pro_Wout])
    bout = jnp.stack([mol_bout, pro_bout])[:, None, :]
    f1b = fc1_b[None, :]
    f2b = fc2_b[None, :]
    ob = out_b[None, :]

    ones_tile = jnp.ones((LANE, 128), jnp.float32)

    deg128 = _deg_kernel(npad, ngroups)(ones_tile, dstpad)
    h0, g, dinv = _init_tc(x, w0, b0, deg128, blk)

    prop = _prop_kernel(npad, ngroups)
    h = h0
    for i in range(L):
        t = prop(g.reshape(2 * npad, 128), srcpad, dstpad)
        h, g = _layer_tc(t, h0, h, dinv, w1[:, i], w2[:, i], cb[:, i], blk)

    return _head_tc(h, wout, bout, fc1_W, f1b, fc2_W, f2b, out_W, ob, n, blk_head)


# speedup vs baseline: 9.3698x; 1.0591x over previous
import functools

import jax
import jax.numpy as jnp
from jax import lax
from jax.experimental import pallas as pl
from jax.experimental.pallas import tpu as pltpu
from jax.experimental.pallas import tpu_sc as plsc

L = 3
ALPHA = 0.2
NSUB = 16
LANE = 128
GROUP = 16
TILE = 128
DEGW = 16


def _sc_mesh():
    return plsc.VectorSubcoreMesh(core_axis_name="c", subcore_axis_name="s")


def _deg_kernel(npad, ngroups):
    tiles = npad // (NSUB * TILE)

    @functools.partial(
        pl.kernel,
        out_type=jax.ShapeDtypeStruct((2, npad, 128), jnp.float32),
        mesh=_sc_mesh(),
        scratch_types=[
            pltpu.VMEM_SHARED((npad + 8, 128), jnp.float32),
            pltpu.VMEM((GROUP, LANE), jnp.int32),
            pltpu.VMEM((LANE, 128), jnp.float32),
            pltpu.SemaphoreType.DMA,
        ],
    )
    def deg_kernel(ones_hbm, dstpad, deg_out, deg_sp, dst_v, rows_v, sem):
        c = lax.axis_index("c")
        s = lax.axis_index("s")
        base = s * (tiles * TILE)
        pltpu.sync_copy(ones_hbm, rows_v)
        for k in range(tiles):
            pltpu.sync_copy(rows_v, deg_sp.at[pl.ds(base + k * TILE, TILE)])
        plsc.subcore_barrier()

        def body(grp, carry):
            pltpu.sync_copy(dstpad.at[c, s, grp], dst_v)
            ds_ = [pltpu.async_copy(rows_v, deg_sp.at[dst_v.at[j]], sem,
                                    add=True) for j in range(GROUP)]
            for d in ds_:
                d.wait()
            return carry

        lax.fori_loop(0, ngroups, body, 0)
        plsc.subcore_barrier()
        for k in range(tiles):
            pltpu.sync_copy(deg_sp.at[pl.ds(base + k * TILE, TILE)], rows_v)
            pltpu.sync_copy(rows_v, deg_out.at[c, pl.ds(base + k * TILE, TILE)])

    return deg_kernel


def _prop_kernel(npad, ngroups):
    tiles = npad // (NSUB * TILE)

    @functools.partial(
        pl.kernel,
        out_type=jax.ShapeDtypeStruct((2, npad, 128), jnp.float32),
        mesh=_sc_mesh(),
        scratch_types=[
            pltpu.VMEM_SHARED((npad + 8, 128), jnp.float32),
            pltpu.VMEM((GROUP, LANE), jnp.int32),
            pltpu.VMEM((GROUP, LANE), jnp.int32),
            pltpu.VMEM((LANE, 128), jnp.float32),
            pltpu.VMEM((LANE, 128), jnp.float32),
            pltpu.SemaphoreType.DMA,
            pltpu.SemaphoreType.DMA,
            pltpu.SemaphoreType.DMA,
            pltpu.SemaphoreType.DMA,
        ],
    )
    def prop_kernel(gflat, srcpad, dstpad, t_out, acc, src_v, dst_v,
                    rows0, rows1, gsem0, gsem1, ssem0, ssem1):
        c = lax.axis_index("c")
        s = lax.axis_index("s")
        base = s * (tiles * TILE)
        rows = (rows0, rows1)
        gsem = (gsem0, gsem1)
        ssem = (ssem0, ssem1)

        ind = [None] * tiles
        outd = [None] * tiles
        ind[0] = pltpu.async_copy(
            gflat.at[pl.ds(c * npad + base, TILE)], rows[0], gsem[0])
        for k in range(tiles):
            ind[k].wait()
            if k >= 1:
                outd[k - 1].wait()
            if k + 1 < tiles:
                b = (k + 1) % 2
                ind[k + 1] = pltpu.async_copy(
                    gflat.at[pl.ds(c * npad + base + (k + 1) * TILE, TILE)],
                    rows[b], gsem[b])
            b = k % 2
            outd[k] = pltpu.async_copy(
                rows[b], acc.at[pl.ds(base + k * TILE, TILE)], ssem[b])
        outd[tiles - 1].wait()
        plsc.subcore_barrier()

        def body(grp, carry):
            pltpu.sync_copy(srcpad.at[c, s, grp], src_v)
            pltpu.sync_copy(dstpad.at[c, s, grp], dst_v)
            gd = [None] * GROUP
            sd = [None] * GROUP
            gd[0] = pltpu.async_copy(gflat.at[src_v.at[0]], rows[0], gsem[0])
            for j in range(GROUP):
                if j >= 1:
                    sd[j - 1].wait()
                if j + 1 < GROUP:
                    b = (j + 1) % 2
                    gd[j + 1] = pltpu.async_copy(
                        gflat.at[src_v.at[j + 1]], rows[b], gsem[b])
                gd[j].wait()
                b = j % 2
                sd[j] = pltpu.async_copy(
                    rows[b], acc.at[dst_v.at[j]], ssem[b], add=True)
            sd[GROUP - 1].wait()
            return carry

        lax.fori_loop(0, ngroups, body, 0)
        plsc.subcore_barrier()

        ind = [None] * tiles
        outd = [None] * tiles
        ind[0] = pltpu.async_copy(acc.at[pl.ds(base, TILE)], rows[0], gsem[0])
        for k in range(tiles):
            ind[k].wait()
            if k >= 1:
                outd[k - 1].wait()
            if k + 1 < tiles:
                b = (k + 1) % 2
                ind[k + 1] = pltpu.async_copy(
                    acc.at[pl.ds(base + (k + 1) * TILE, TILE)], rows[b], gsem[b])
            b = k % 2
            outd[k] = pltpu.async_copy(
                rows[b], t_out.at[c, pl.ds(base + k * TILE, TILE)], ssem[b])
        outd[tiles - 1].wait()

    return prop_kernel


_F32 = jnp.float32


def _init_tc(x, w0, b0, deg, blk):
    n = x.shape[1]
    grid = (2, n // blk)
    out = [
        jax.ShapeDtypeStruct((2, n, 128), _F32),
        jax.ShapeDtypeStruct((2, n, 128), _F32),
        jax.ShapeDtypeStruct((2, n, 1), _F32),
    ]

    def body(x_ref, w_ref, b_ref, deg_ref, h_ref, g_ref, dinv_ref):
        h = jnp.dot(x_ref[0], w_ref[0], preferred_element_type=_F32) + b_ref[0]
        h = jnp.maximum(h, 0.0)
        dinv = lax.rsqrt(deg_ref[0][:, 0:1])
        h_ref[0] = h
        g_ref[0] = h * dinv
        dinv_ref[0] = dinv

    return pl.pallas_call(
        body,
        grid=grid,
        in_specs=[
            pl.BlockSpec((1, blk, 128), lambda c, b: (c, b, 0)),
            pl.BlockSpec((1, 128, 128), lambda c, b: (c, 0, 0)),
            pl.BlockSpec((1, 1, 128), lambda c, b: (c, 0, 0)),
            pl.BlockSpec((1, blk, 128), lambda c, b: (c, b, 0)),
        ],
        out_specs=[
            pl.BlockSpec((1, blk, 128), lambda c, b: (c, b, 0)),
            pl.BlockSpec((1, blk, 128), lambda c, b: (c, b, 0)),
            pl.BlockSpec((1, blk, 1), lambda c, b: (c, b, 0)),
        ],
        out_shape=out,
    )(x, w0, b0, deg)


def _layer_tc(t, h0, h, dinv, w1, w2, cb, blk):
    n = t.shape[1]
    grid = (2, n // blk)
    out = [
        jax.ShapeDtypeStruct((2, n, 128), _F32),
        jax.ShapeDtypeStruct((2, n, 128), _F32),
    ]

    def body(t_ref, h0_ref, h_ref, dinv_ref, w1_ref, w2_ref, cb_ref, hn_ref, gn_ref):
        dinv = dinv_ref[0]
        agg = t_ref[0] * dinv
        o = (
            jnp.dot(agg, w1_ref[0], preferred_element_type=_F32)
            + jnp.dot(h0_ref[0], w2_ref[0], preferred_element_type=_F32)
            + cb_ref[0]
        )
        hn = jnp.maximum(o, 0.0) + h_ref[0]
        hn_ref[0] = hn
        gn_ref[0] = hn * dinv

    return pl.pallas_call(
        body,
        grid=grid,
        in_specs=[
            pl.BlockSpec((1, blk, 128), lambda c, b: (c, b, 0)),
            pl.BlockSpec((1, blk, 128), lambda c, b: (c, b, 0)),
            pl.BlockSpec((1, blk, 128), lambda c, b: (c, b, 0)),
            pl.BlockSpec((1, blk, 1), lambda c, b: (c, b, 0)),
            pl.BlockSpec((1, 128, 128), lambda c, b: (c, 0, 0)),
            pl.BlockSpec((1, 128, 128), lambda c, b: (c, 0, 0)),
            pl.BlockSpec((1, 1, 128), lambda c, b: (c, 0, 0)),
        ],
        out_specs=[
            pl.BlockSpec((1, blk, 128), lambda c, b: (c, b, 0)),
            pl.BlockSpec((1, blk, 128), lambda c, b: (c, b, 0)),
        ],
        out_shape=out,
    )(t, h0, h, dinv, w1, w2, cb)


def _head_tc(h, wout, bout, fc1w, fc1b, fc2w, fc2b, outw, outb, n, blk):
    grid = (n // blk,)

    def body(h_ref, wout_ref, bout_ref, f1w_ref, f1b_ref, f2w_ref, f2b_ref,
             ow_ref, ob_ref, o_ref):
        ym = jnp.dot(h_ref[0], wout_ref[0], preferred_element_type=_F32) + bout_ref[0]
        yp = jnp.dot(h_ref[1], wout_ref[1], preferred_element_type=_F32) + bout_ref[1]
        t1 = (
            jnp.dot(ym, f1w_ref[0:128, :], preferred_element_type=_F32)
            + jnp.dot(yp, f1w_ref[128:256, :], preferred_element_type=_F32)
            + f1b_ref[...]
        )
        t1 = jnp.maximum(t1, 0.0)
        t2 = jnp.maximum(
            jnp.dot(t1, f2w_ref[...], preferred_element_type=_F32) + f2b_ref[...], 0.0)
        o_ref[...] = jnp.dot(t2, ow_ref[...], preferred_element_type=_F32) + ob_ref[...]

    return pl.pallas_call(
        body,
        grid=grid,
        in_specs=[
            pl.BlockSpec((2, blk, 128), lambda b: (0, b, 0)),
            pl.BlockSpec((2, 128, 128), lambda b: (0, 0, 0)),
            pl.BlockSpec((2, 1, 128), lambda b: (0, 0, 0)),
            pl.BlockSpec((256, 1024), lambda b: (0, 0)),
            pl.BlockSpec((1, 1024), lambda b: (0, 0)),
            pl.BlockSpec((1024, 512), lambda b: (0, 0)),
            pl.BlockSpec((1, 512), lambda b: (0, 0)),
            pl.BlockSpec((512, 1), lambda b: (0, 0)),
            pl.BlockSpec((1, 1), lambda b: (0, 0)),
        ],
        out_specs=pl.BlockSpec((blk, 1), lambda b: (b, 0)),
        out_shape=jax.ShapeDtypeStruct((n, 1), _F32),
    )(h, wout, bout, fc1w, fc1b, fc2w, fc2b, outw, outb)


def _prep_edges(ei, npad, offset, ngroups):
    src = ei[0].astype(jnp.int32)
    dst = ei[1].astype(jnp.int32)
    e = src.shape[0]
    total = NSUB * ngroups * GROUP * LANE
    pad = total - e
    src = jnp.concatenate([src + offset, jnp.full((pad,), offset, jnp.int32)])
    dst = jnp.concatenate([dst, jnp.full((pad,), npad, jnp.int32)])
    return (src.reshape(NSUB, ngroups, GROUP, LANE),
            dst.reshape(NSUB, ngroups, GROUP, LANE))


def _pad_feat(x, w, npad):
    n, f = x.shape
    x = jnp.pad(x, ((0, npad - n), (0, 128 - f)))
    w = jnp.pad(w, ((0, 128 - f), (0, 0)))
    return x, w


def kernel(mol_x, pro_x, mol_edge_index, pro_edge_index, mol_batch, pro_batch,
           mol_W0, mol_b0, mol_W1s, mol_W2s, mol_cbs, mol_Wout, mol_bout,
           pro_W0, pro_b0, pro_W1s, pro_W2s, pro_cbs, pro_Wout, pro_bout,
           fc1_W, fc1_b, fc2_W, fc2_b, out_W, out_b):
    n = mol_x.shape[0]
    e = mol_edge_index.shape[1]
    npad = -(-n // (NSUB * TILE)) * NSUB * TILE
    nchunks = -(-e // (NSUB * LANE))
    ngroups = -(-nchunks // GROUP)
    nchunks_pad = ngroups * GROUP
    blk = npad // 8
    blk_head = 1000 if n % 1000 == 0 else n

    mx, mW0 = _pad_feat(mol_x, mol_W0, npad)
    px, pW0 = _pad_feat(pro_x, pro_W0, npad)
    x = jnp.stack([mx, px])
    w0 = jnp.stack([mW0, pW0])
    b0 = jnp.stack([mol_b0, pro_b0])[:, None, :]

    msrc, mdst = _prep_edges(mol_edge_index, npad, 0, ngroups)
    psrc, pdst = _prep_edges(pro_edge_index, npad, npad, ngroups)
    srcpad = jnp.stack([msrc, psrc])
    dstpad = jnp.stack([mdst, pdst])

    w1 = jnp.stack([mol_W1s, pro_W1s]) * (1.0 - ALPHA)
    w2 = jnp.stack([mol_W2s, pro_W2s]) * ALPHA
    cb = jnp.stack([mol_cbs, pro_cbs])[:, :, None, :]
    wout = jnp.stack([mol_Wout, pro_Wout])
    bout = jnp.stack([mol_bout, pro_bout])[:, None, :]
    f1b = fc1_b[None, :]
    f2b = fc2_b[None, :]
    ob = out_b[None, :]

    ones_tile = jnp.ones((LANE, 128), jnp.float32)

    deg128 = _deg_kernel(npad, ngroups)(ones_tile, dstpad)
    h0, g, dinv = _init_tc(x, w0, b0, deg128, blk)

    prop = _prop_kernel(npad, ngroups)
    h = h0
    for i in range(L):
        t = prop(g.reshape(2 * npad, 128), srcpad, dstpad)
        h, g = _layer_tc(t, h0, h, dinv, w1[:, i], w2[:, i], cb[:, i], blk)

    return _head_tc(h, wout, bout, fc1_W, f1b, fc2_W, f2b, out_W, ob, n, blk_head)
